# baseline (speedup 1.0000x reference)
import functools

import jax
import jax.numpy as jnp
from jax import lax
from jax.experimental import pallas as pl
from jax.experimental.pallas import tpu as pltpu
from jax.experimental.pallas import tpu_sc as plsc

N = 10000
IN_CH = 256
HID_CH = 32
N_EDGES = 160000

NC, NS = 2, 16
NW = NC * NS
RPT = N // NS
K = 128
CHUNKS = 40
EPAD = NW * CHUNKS * K
NSP = N + 16
NBUF = 4

_mesh = plsc.VectorSubcoreMesh(core_axis_name="c", subcore_axis_name="s")
_sc_params = pltpu.CompilerParams(use_tc_tiling_on_sc=False)


@functools.partial(
    pl.kernel,
    mesh=_mesh,
    out_type=jax.ShapeDtypeStruct((NC * N, HID_CH), jnp.float32),
    scratch_types=[
        pltpu.VMEM((CHUNKS, K), jnp.int32),
        pltpu.VMEM((K, HID_CH), jnp.float32),
        pltpu.VMEM_SHARED((NSP, HID_CH), jnp.float32),
        pltpu.SemaphoreType.DMA,
    ],
    compiler_params=_sc_params,
)
def _sc_deg(dst_hbm, zeros_hbm, ones_hbm, out_hbm, dstv, onesv, deg_sh, sem):
    c = lax.axis_index("c")
    s = lax.axis_index("s")
    wid = s * NC + c
    r0 = s * RPT
    pltpu.sync_copy(zeros_hbm.at[pl.ds(r0, RPT)], deg_sh.at[pl.ds(r0, RPT)])
    pltpu.sync_copy(ones_hbm, onesv)
    pltpu.sync_copy(dst_hbm.at[pl.ds(wid * CHUNKS, CHUNKS)], dstv)
    plsc.subcore_barrier()

    def fire(j, carry):
        pltpu.async_copy(onesv, deg_sh.at[dstv.at[j]], sem, add=True)
        return carry

    lax.fori_loop(0, CHUNKS, fire, 0)

    def drain(j, carry):
        pltpu.make_async_copy(onesv, deg_sh.at[dstv.at[0]], sem).wait()
        return carry

    lax.fori_loop(0, CHUNKS, drain, 0)
    plsc.subcore_barrier()
    pltpu.sync_copy(deg_sh.at[pl.ds(r0, RPT)],
                    out_hbm.at[pl.ds(c * N + r0, RPT)])


@functools.partial(
    pl.kernel,
    mesh=_mesh,
    out_type=jax.ShapeDtypeStruct((NC * N, HID_CH), jnp.float32),
    scratch_types=[
        pltpu.VMEM((CHUNKS, K), jnp.int32),
        pltpu.VMEM((CHUNKS, K), jnp.int32),
        pltpu.VMEM((NBUF, K, HID_CH), jnp.float32),
        pltpu.VMEM_SHARED((NSP, HID_CH), jnp.float32),
    ] + [pltpu.SemaphoreType.DMA] * (2 * NBUF),
    compiler_params=_sc_params,
)
def _sc_agg(src_hbm, dst_hbm, feat_hbm, zeros_hbm, out_hbm,
            srcv, dstv, rows, agg_sh, *sems):
    gsem = sems[:NBUF]
    ssem = sems[NBUF:]
    c = lax.axis_index("c")
    s = lax.axis_index("s")
    wid = s * NC + c
    r0 = s * RPT
    pltpu.sync_copy(zeros_hbm.at[pl.ds(r0, RPT)], agg_sh.at[pl.ds(r0, RPT)])
    pltpu.sync_copy(src_hbm.at[pl.ds(wid * CHUNKS, CHUNKS)], srcv)
    pltpu.sync_copy(dst_hbm.at[pl.ds(wid * CHUNKS, CHUNKS)], dstv)
    plsc.subcore_barrier()

    pltpu.async_copy(feat_hbm.at[srcv.at[0]], rows.at[0], gsem[0])
    pltpu.async_copy(feat_hbm.at[srcv.at[1]], rows.at[1], gsem[1])

    def body(i, carry):
        for b in range(NBUF):
            t = i * NBUF + b
            b2 = (b + 2) % NBUF
            pltpu.make_async_copy(feat_hbm.at[srcv.at[t]], rows.at[b],
                                  gsem[b]).wait()
            pltpu.async_copy(rows.at[b], agg_sh.at[dstv.at[t]], ssem[b],
                             add=True)

            @pl.when(t >= 2)
            def _():
                pltpu.make_async_copy(rows.at[b2], agg_sh.at[dstv.at[t]],
                                      ssem[b2]).wait()

            @pl.when(t + 2 < CHUNKS)
            def _():
                pltpu.async_copy(feat_hbm.at[srcv.at[t + 2]], rows.at[b2],
                                 gsem[b2])
        return carry

    lax.fori_loop(0, CHUNKS // NBUF, body, 0)
    pltpu.make_async_copy(rows.at[2], agg_sh.at[dstv.at[0]], ssem[2]).wait()
    pltpu.make_async_copy(rows.at[3], agg_sh.at[dstv.at[0]], ssem[3]).wait()
    plsc.subcore_barrier()
    pltpu.sync_copy(agg_sh.at[pl.ds(r0, RPT)],
                    out_hbm.at[pl.ds(c * N + r0, RPT)])


NV = N // 4
OUT_W = 8


def _tc1_body(x_ref, w_ref, dg_ref, hs_ref, dinv_ref):
    deg = dg_ref[:NV] + dg_ref[NV:] + 1.0
    dinv = lax.rsqrt(deg)
    h = jnp.dot(x_ref[...], w_ref[...], preferred_element_type=jnp.float32)
    hs_ref[...] = h * dinv
    dinv_ref[...] = dinv


def _tc2_body(ag_ref, hs_ref, dinv_ref, b1_ref, zs_ref):
    dinv = dinv_ref[...]
    t = (ag_ref[:NV] + ag_ref[NV:] + hs_ref[...]) * dinv + b1_ref[...]
    zs_ref[...] = jnp.maximum(t, 0.0) * dinv


def _tc3_body(ag_ref, zs_ref, dinv_ref, w2_ref, b2_ref, out_ref):
    u = (ag_ref[:NV] + ag_ref[NV:] + zs_ref[...]) * dinv_ref[...]
    g = jnp.dot(u, w2_ref[...], preferred_element_type=jnp.float32)
    out_ref[...] = g + b2_ref[...]


def kernel(x, edge_index, W1, b1, W2, b2):
    ei = edge_index.astype(jnp.int32)
    npad = EPAD - N_EDGES
    src = jnp.concatenate([ei[0], jnp.zeros((npad,), jnp.int32)])
    dst = jnp.concatenate(
        [ei[1], N + (jnp.arange(npad, dtype=jnp.int32) % 16)])
    src = src.reshape(NW * CHUNKS, K)
    dst = dst.reshape(NW * CHUNKS, K)

    zeros_w = jnp.zeros((N, HID_CH), jnp.float32)
    ones_k = jnp.ones((K, HID_CH), jnp.float32)
    b1t = jnp.tile(b1, 4).reshape(1, 128)
    eye4 = jnp.eye(4, dtype=jnp.float32)
    w1big = jnp.kron(eye4, W1)
    w2p = jnp.zeros((HID_CH, OUT_W), jnp.float32).at[:, :2].set(W2)
    w2big = jnp.kron(eye4, w2p)
    b2t = jnp.tile(b2p := jnp.zeros((OUT_W,), jnp.float32).at[:2].set(b2),
                   4).reshape(1, 32)
    xv = x.reshape(NV, 4 * IN_CH)

    deg2 = _sc_deg(dst, zeros_w, ones_k)
    deg2v = deg2.reshape(NC * N // 4, 128)

    hs, dinv = pl.pallas_call(
        _tc1_body,
        out_shape=[jax.ShapeDtypeStruct((NV, 128), jnp.float32),
                   jax.ShapeDtypeStruct((NV, 128), jnp.float32)],
    )(xv, w1big, deg2v)

    agg1 = _sc_agg(src, dst, hs.reshape(N, HID_CH), zeros_w)
    agg1v = agg1.reshape(NC * N // 4, 128)

    zs = pl.pallas_call(
        _tc2_body,
        out_shape=jax.ShapeDtypeStruct((NV, 128), jnp.float32),
    )(agg1v, hs, dinv, b1t)

    agg2 = _sc_agg(src, dst, zs.reshape(N, HID_CH), zeros_w)
    agg2v = agg2.reshape(NC * N // 4, 128)

    outp = pl.pallas_call(
        _tc3_body,
        out_shape=jax.ShapeDtypeStruct((NV, 4 * OUT_W), jnp.float32),
    )(agg2v, zs, dinv, w2big, b2t)

    return outp.reshape(N, OUT_W)[:, :2]


# speedup vs baseline: 1.0628x; 1.0628x over previous
import functools

import jax
import jax.numpy as jnp
from jax import lax
from jax.experimental import pallas as pl
from jax.experimental.pallas import tpu as pltpu
from jax.experimental.pallas import tpu_sc as plsc

N = 10000
IN_CH = 256
HID_CH = 32
N_EDGES = 160000

NC, NS = 2, 16
NW = NC * NS
RPT = N // NS
K = 128
CHUNKS = 40
EPAD = NW * CHUNKS * K
NSP = N + 4096
NBUF = 4

_mesh = plsc.VectorSubcoreMesh(core_axis_name="c", subcore_axis_name="s")
_sc_params = pltpu.CompilerParams(use_tc_tiling_on_sc=False)


@functools.partial(
    pl.kernel,
    mesh=_mesh,
    out_type=jax.ShapeDtypeStruct((NC * N, HID_CH), jnp.float32),
    scratch_types=[
        pltpu.VMEM((CHUNKS, K), jnp.int32),
        pltpu.VMEM((K, HID_CH), jnp.float32),
        pltpu.VMEM_SHARED((NSP, HID_CH), jnp.float32),
        pltpu.SemaphoreType.DMA,
    ],
    compiler_params=_sc_params,
)
def _sc_deg(dst_hbm, zeros_hbm, ones_hbm, out_hbm, dstv, onesv, deg_sh, sem):
    c = lax.axis_index("c")
    s = lax.axis_index("s")
    wid = s * NC + c
    r0 = s * RPT
    pltpu.sync_copy(zeros_hbm.at[pl.ds(r0, RPT)], deg_sh.at[pl.ds(r0, RPT)])
    pltpu.sync_copy(ones_hbm, onesv)
    pltpu.sync_copy(dst_hbm.at[pl.ds(wid * CHUNKS, CHUNKS)], dstv)
    plsc.subcore_barrier()

    def fire(j, carry):
        pltpu.async_copy(onesv, deg_sh.at[dstv.at[j]], sem, add=True)
        return carry

    lax.fori_loop(0, CHUNKS, fire, 0)

    def drain(j, carry):
        pltpu.make_async_copy(onesv, deg_sh.at[dstv.at[0]], sem).wait()
        return carry

    lax.fori_loop(0, CHUNKS, drain, 0)
    plsc.subcore_barrier()
    pltpu.sync_copy(deg_sh.at[pl.ds(r0, RPT)],
                    out_hbm.at[pl.ds(c * N + r0, RPT)])


@functools.partial(
    pl.kernel,
    mesh=_mesh,
    out_type=jax.ShapeDtypeStruct((NC * N, HID_CH), jnp.float32),
    scratch_types=[
        pltpu.VMEM((CHUNKS, K), jnp.int32),
        pltpu.VMEM((CHUNKS, K), jnp.int32),
        pltpu.VMEM((NBUF, K, HID_CH), jnp.float32),
        pltpu.VMEM_SHARED((NSP, HID_CH), jnp.float32),
    ] + [pltpu.SemaphoreType.DMA] * (2 * NBUF),
    compiler_params=_sc_params,
)
def _sc_agg(src_hbm, dst_hbm, feat_hbm, zeros_hbm, out_hbm,
            srcv, dstv, rows, agg_sh, *sems):
    gsem = sems[:NBUF]
    ssem = sems[NBUF:]
    c = lax.axis_index("c")
    s = lax.axis_index("s")
    wid = s * NC + c
    r0 = s * RPT
    pltpu.sync_copy(zeros_hbm.at[pl.ds(r0, RPT)], agg_sh.at[pl.ds(r0, RPT)])
    pltpu.sync_copy(src_hbm.at[pl.ds(wid * CHUNKS, CHUNKS)], srcv)
    pltpu.sync_copy(dst_hbm.at[pl.ds(wid * CHUNKS, CHUNKS)], dstv)
    plsc.subcore_barrier()

    pltpu.async_copy(feat_hbm.at[srcv.at[0]], rows.at[0], gsem[0])
    pltpu.async_copy(feat_hbm.at[srcv.at[1]], rows.at[1], gsem[1])

    def body(i, carry):
        for b in range(NBUF):
            t = i * NBUF + b
            b2 = (b + 2) % NBUF
            pltpu.make_async_copy(feat_hbm.at[srcv.at[t]], rows.at[b],
                                  gsem[b]).wait()
            pltpu.async_copy(rows.at[b], agg_sh.at[dstv.at[t]], ssem[b],
                             add=True)

            @pl.when(t >= 2)
            def _():
                pltpu.make_async_copy(rows.at[b2], agg_sh.at[dstv.at[t]],
                                      ssem[b2]).wait()

            @pl.when(t + 2 < CHUNKS)
            def _():
                pltpu.async_copy(feat_hbm.at[srcv.at[t + 2]], rows.at[b2],
                                 gsem[b2])
        return carry

    lax.fori_loop(0, CHUNKS // NBUF, body, 0)
    pltpu.make_async_copy(rows.at[2], agg_sh.at[dstv.at[0]], ssem[2]).wait()
    pltpu.make_async_copy(rows.at[3], agg_sh.at[dstv.at[0]], ssem[3]).wait()
    plsc.subcore_barrier()
    pltpu.sync_copy(agg_sh.at[pl.ds(r0, RPT)],
                    out_hbm.at[pl.ds(c * N + r0, RPT)])


NV = N // 4
OUT_W = 8


def _tc1_body(x_ref, w_ref, dg_ref, hs_ref, dinv_ref):
    deg = dg_ref[:NV] + dg_ref[NV:] + 1.0
    dinv = lax.rsqrt(deg)
    w = w_ref[...]
    h = jnp.concatenate(
        [jnp.dot(x_ref[:, a * IN_CH:(a + 1) * IN_CH], w,
                 preferred_element_type=jnp.float32) for a in range(4)],
        axis=1)
    hs_ref[...] = h * dinv
    dinv_ref[...] = dinv


def _tc2_body(ag_ref, hs_ref, dinv_ref, b1_ref, zs_ref):
    dinv = dinv_ref[...]
    t = (ag_ref[:NV] + ag_ref[NV:] + hs_ref[...]) * dinv + b1_ref[...]
    zs_ref[...] = jnp.maximum(t, 0.0) * dinv


def _tc3_body(ag_ref, zs_ref, dinv_ref, w2_ref, b2_ref, out_ref):
    u = (ag_ref[:NV] + ag_ref[NV:] + zs_ref[...]) * dinv_ref[...]
    w2 = w2_ref[...]
    g = jnp.concatenate(
        [jnp.dot(u[:, a * HID_CH:(a + 1) * HID_CH], w2,
                 preferred_element_type=jnp.float32) for a in range(4)],
        axis=1)
    out_ref[...] = g + b2_ref[...]


def kernel(x, edge_index, W1, b1, W2, b2):
    ei = edge_index.astype(jnp.int32)
    npad = EPAD - N_EDGES
    src = jnp.concatenate([ei[0], jnp.zeros((npad,), jnp.int32)])
    dst = jnp.concatenate(
        [ei[1], N + (jnp.arange(npad, dtype=jnp.int32) % 4096)])
    src = src.reshape(NW * CHUNKS, K)
    dst = dst.reshape(NW * CHUNKS, K)

    zeros_w = jnp.zeros((N, HID_CH), jnp.float32)
    ones_k = jnp.ones((K, HID_CH), jnp.float32)
    b1t = jnp.tile(b1, 4).reshape(1, 128)
    w2p = jnp.zeros((HID_CH, OUT_W), jnp.float32).at[:, :2].set(W2)
    b2t = jnp.tile(jnp.zeros((OUT_W,), jnp.float32).at[:2].set(b2),
                   4).reshape(1, 32)
    xv = x.reshape(NV, 4 * IN_CH)

    deg2 = _sc_deg(dst, zeros_w, ones_k)
    deg2v = deg2.reshape(NC * N // 4, 128)

    hs, dinv = pl.pallas_call(
        _tc1_body,
        out_shape=[jax.ShapeDtypeStruct((NV, 128), jnp.float32),
                   jax.ShapeDtypeStruct((NV, 128), jnp.float32)],
    )(xv, W1, deg2v)

    agg1 = _sc_agg(src, dst, hs.reshape(N, HID_CH), zeros_w)
    agg1v = agg1.reshape(NC * N // 4, 128)

    zs = pl.pallas_call(
        _tc2_body,
        out_shape=jax.ShapeDtypeStruct((NV, 128), jnp.float32),
    )(agg1v, hs, dinv, b1t)

    agg2 = _sc_agg(src, dst, zs.reshape(N, HID_CH), zeros_w)
    agg2v = agg2.reshape(NC * N // 4, 128)

    outp = pl.pallas_call(
        _tc3_body,
        out_shape=jax.ShapeDtypeStruct((NV, 4 * OUT_W), jnp.float32),
    )(agg2v, zs, dinv, w2p, b2t)

    return outp.reshape(N, OUT_W)[:, :2]


# speedup vs baseline: 1.6634x; 1.5651x over previous
import functools

import jax
import jax.numpy as jnp
from jax import lax
from jax.experimental import pallas as pl
from jax.experimental.pallas import tpu as pltpu
from jax.experimental.pallas import tpu_sc as plsc

N = 10000
IN_CH = 256
HID_CH = 32
N_EDGES = 160000

NC, NS = 2, 16
NW = NC * NS
RPT = N // NS
K = 128
CHUNKS = 40
EPAD = NW * CHUNKS * K
NSP = N + 4096
NBUF = 4

_mesh = plsc.VectorSubcoreMesh(core_axis_name="c", subcore_axis_name="s")
_sc_params = pltpu.CompilerParams(use_tc_tiling_on_sc=False)


@functools.partial(
    pl.kernel,
    mesh=_mesh,
    out_type=jax.ShapeDtypeStruct((NC * N, HID_CH), jnp.float32),
    scratch_types=[
        pltpu.VMEM((CHUNKS, K), jnp.int32),
        pltpu.VMEM((K, HID_CH), jnp.float32),
        pltpu.VMEM_SHARED((NSP, HID_CH), jnp.float32),
        pltpu.SemaphoreType.DMA,
    ],
    compiler_params=_sc_params,
)
def _sc_deg(dst_hbm, zeros_hbm, ones_hbm, out_hbm, dstv, onesv, deg_sh, sem):
    c = lax.axis_index("c")
    s = lax.axis_index("s")
    wid = s * NC + c
    r0 = s * RPT
    pltpu.sync_copy(zeros_hbm.at[pl.ds(r0, RPT)], deg_sh.at[pl.ds(r0, RPT)])
    pltpu.sync_copy(ones_hbm, onesv)
    pltpu.sync_copy(dst_hbm.at[pl.ds(wid * CHUNKS, CHUNKS)], dstv)
    plsc.subcore_barrier()

    def fire(j, carry):
        pltpu.async_copy(onesv, deg_sh.at[dstv.at[j]], sem, add=True)
        return carry

    lax.fori_loop(0, CHUNKS, fire, 0)

    def drain(j, carry):
        pltpu.make_async_copy(onesv, deg_sh.at[dstv.at[0]], sem).wait()
        return carry

    lax.fori_loop(0, CHUNKS, drain, 0)
    plsc.subcore_barrier()
    pltpu.sync_copy(deg_sh.at[pl.ds(r0, RPT)],
                    out_hbm.at[pl.ds(c * N + r0, RPT)])


@functools.partial(
    pl.kernel,
    mesh=_mesh,
    out_type=jax.ShapeDtypeStruct((NC * N, HID_CH), jnp.float32),
    scratch_types=[
        pltpu.VMEM((CHUNKS, K), jnp.int32),
        pltpu.VMEM((CHUNKS, K), jnp.int32),
        pltpu.VMEM((NBUF, K, HID_CH), jnp.float32),
        pltpu.VMEM_SHARED((NSP, HID_CH), jnp.float32),
    ] + [pltpu.SemaphoreType.DMA] * (2 * NBUF),
    compiler_params=_sc_params,
)
def _sc_agg(src_hbm, dst_hbm, feat_hbm, zeros_hbm, out_hbm,
            srcv, dstv, rows, agg_sh, *sems):
    gsem = sems[:NBUF]
    ssem = sems[NBUF:]
    c = lax.axis_index("c")
    s = lax.axis_index("s")
    wid = s * NC + c
    r0 = s * RPT
    pltpu.sync_copy(zeros_hbm.at[pl.ds(r0, RPT)], agg_sh.at[pl.ds(r0, RPT)])
    pltpu.sync_copy(src_hbm.at[pl.ds(wid * CHUNKS, CHUNKS)], srcv)
    pltpu.sync_copy(dst_hbm.at[pl.ds(wid * CHUNKS, CHUNKS)], dstv)
    plsc.subcore_barrier()

    pltpu.async_copy(feat_hbm.at[srcv.at[0]], rows.at[0], gsem[0])
    pltpu.async_copy(feat_hbm.at[srcv.at[1]], rows.at[1], gsem[1])

    def body(i, carry):
        for b in range(NBUF):
            t = i * NBUF + b
            b2 = (b + 2) % NBUF
            pltpu.make_async_copy(feat_hbm.at[srcv.at[t]], rows.at[b],
                                  gsem[b]).wait()
            pltpu.async_copy(rows.at[b], agg_sh.at[dstv.at[t]], ssem[b],
                             add=True)

            @pl.when(t >= 2)
            def _():
                pltpu.make_async_copy(rows.at[b2], agg_sh.at[dstv.at[t]],
                                      ssem[b2]).wait()

            @pl.when(t + 2 < CHUNKS)
            def _():
                pltpu.async_copy(feat_hbm.at[srcv.at[t + 2]], rows.at[b2],
                                 gsem[b2])
        return carry

    lax.fori_loop(0, CHUNKS // NBUF, body, 0)
    pltpu.make_async_copy(rows.at[2], agg_sh.at[dstv.at[0]], ssem[2]).wait()
    pltpu.make_async_copy(rows.at[3], agg_sh.at[dstv.at[0]], ssem[3]).wait()
    plsc.subcore_barrier()
    pltpu.sync_copy(agg_sh.at[pl.ds(r0, RPT)],
                    out_hbm.at[pl.ds(c * N + r0, RPT)])


NV = N // 4
OUT_W = 8


def _tc1_body(x_ref, w_ref, dg_ref, hs_ref, dinv_ref):
    deg = dg_ref[:NV] + dg_ref[NV:] + 1.0
    dinv = lax.rsqrt(deg)
    w = w_ref[...]
    h = jnp.concatenate(
        [jnp.dot(x_ref[:, a * IN_CH:(a + 1) * IN_CH], w,
                 preferred_element_type=jnp.float32) for a in range(4)],
        axis=1)
    hs_ref[...] = h * dinv
    dinv_ref[...] = dinv


def _tc2_body(ag_ref, hs_ref, dinv_ref, b1_ref, zs_ref):
    dinv = dinv_ref[...]
    t = (ag_ref[:NV] + ag_ref[NV:] + hs_ref[...]) * dinv + b1_ref[...]
    zs_ref[...] = jnp.maximum(t, 0.0) * dinv


def _tc3_body(ag_ref, zs_ref, dinv_ref, w2_ref, b2_ref, out_ref):
    u = (ag_ref[:NV] + ag_ref[NV:] + zs_ref[...]) * dinv_ref[...]
    w2 = w2_ref[...]
    g = jnp.concatenate(
        [jnp.dot(u[:, a * HID_CH:(a + 1) * HID_CH], w2,
                 preferred_element_type=jnp.float32) for a in range(4)],
        axis=1)
    out_ref[...] = g + b2_ref[...]


def kernel(x, edge_index, W1, b1, W2, b2):
    ei = edge_index.astype(jnp.int32)
    npad = EPAD - N_EDGES
    src = jnp.concatenate(
        [ei[0], jnp.arange(npad, dtype=jnp.int32) % 8192])
    dst = jnp.concatenate(
        [ei[1], N + (jnp.arange(npad, dtype=jnp.int32) % 4096)])
    src = src.reshape(NW * CHUNKS, K)
    dst = dst.reshape(NW * CHUNKS, K)

    zeros_w = jnp.zeros((N, HID_CH), jnp.float32)
    ones_k = jnp.ones((K, HID_CH), jnp.float32)
    b1t = jnp.tile(b1, 4).reshape(1, 128)
    w2p = jnp.zeros((HID_CH, OUT_W), jnp.float32).at[:, :2].set(W2)
    b2t = jnp.tile(jnp.zeros((OUT_W,), jnp.float32).at[:2].set(b2),
                   4).reshape(1, 32)
    xv = x.reshape(NV, 4 * IN_CH)

    deg2 = _sc_deg(dst, zeros_w, ones_k)
    deg2v = deg2.reshape(NC * N // 4, 128)

    hs, dinv = pl.pallas_call(
        _tc1_body,
        out_shape=[jax.ShapeDtypeStruct((NV, 128), jnp.float32),
                   jax.ShapeDtypeStruct((NV, 128), jnp.float32)],
    )(xv, W1, deg2v)

    agg1 = _sc_agg(src, dst, hs.reshape(N, HID_CH), zeros_w)
    agg1v = agg1.reshape(NC * N // 4, 128)

    zs = pl.pallas_call(
        _tc2_body,
        out_shape=jax.ShapeDtypeStruct((NV, 128), jnp.float32),
    )(agg1v, hs, dinv, b1t)

    agg2 = _sc_agg(src, dst, zs.reshape(N, HID_CH), zeros_w)
    agg2v = agg2.reshape(NC * N // 4, 128)

    outp = pl.pallas_call(
        _tc3_body,
        out_shape=jax.ShapeDtypeStruct((NV, 4 * OUT_W), jnp.float32),
    )(agg2v, zs, dinv, w2p, b2t)

    return outp.reshape(N, OUT_W)[:, :2]


# speedup vs baseline: 1.6693x; 1.0035x over previous
import functools

import jax
import jax.numpy as jnp
from jax import lax
from jax.experimental import pallas as pl
from jax.experimental.pallas import tpu as pltpu
from jax.experimental.pallas import tpu_sc as plsc

N = 10000
IN_CH = 256
HID_CH = 32
N_EDGES = 160000

NC, NS = 2, 16
NW = NC * NS
RPT = N // NS
K = 128
CHUNKS = 40
EPAD = NW * CHUNKS * K
NSP = N + 4096
NBUF = 4

_mesh = plsc.VectorSubcoreMesh(core_axis_name="c", subcore_axis_name="s")
_sc_params = pltpu.CompilerParams(use_tc_tiling_on_sc=False)


DHW = 16


@functools.partial(
    pl.kernel,
    mesh=_mesh,
    out_type=jax.ShapeDtypeStruct((NC * N, HID_CH), jnp.float32),
    scratch_types=[
        pltpu.VMEM((CHUNKS, K), jnp.int32),
        pltpu.VMEM((K, DHW), jnp.float32),
        pltpu.VMEM_SHARED((NSP, DHW), jnp.float32),
        pltpu.SemaphoreType.DMA,
    ],
    compiler_params=_sc_params,
)
def _sc_deg(dst_hbm, zeros_hbm, ones_hbm, out_hbm, dstv, onesv, deg_sh, sem):
    c = lax.axis_index("c")
    s = lax.axis_index("s")
    wid = s * NC + c
    r0 = s * RPT
    pltpu.sync_copy(zeros_hbm.at[pl.ds(r0, RPT)], deg_sh.at[pl.ds(r0, RPT)])
    pltpu.sync_copy(ones_hbm, onesv)
    pltpu.sync_copy(dst_hbm.at[pl.ds(wid * CHUNKS, CHUNKS)], dstv)
    plsc.subcore_barrier()

    def fire(j, carry):
        pltpu.async_copy(onesv, deg_sh.at[dstv.at[j]], sem, add=True)
        return carry

    lax.fori_loop(0, CHUNKS, fire, 0)

    def drain(j, carry):
        pltpu.make_async_copy(onesv, deg_sh.at[dstv.at[0]], sem).wait()
        return carry

    lax.fori_loop(0, CHUNKS, drain, 0)
    plsc.subcore_barrier()
    pltpu.sync_copy(deg_sh.at[pl.ds(r0, RPT)],
                    out_hbm.at[pl.ds(c * N + r0, RPT), pl.ds(0, DHW)])
    pltpu.sync_copy(deg_sh.at[pl.ds(r0, RPT)],
                    out_hbm.at[pl.ds(c * N + r0, RPT), pl.ds(DHW, DHW)])


@functools.partial(
    pl.kernel,
    mesh=_mesh,
    out_type=jax.ShapeDtypeStruct((NC * N, HID_CH), jnp.float32),
    scratch_types=[
        pltpu.VMEM((CHUNKS, K), jnp.int32),
        pltpu.VMEM((CHUNKS, K), jnp.int32),
        pltpu.VMEM((NBUF, K, HID_CH), jnp.float32),
        pltpu.VMEM_SHARED((NSP, HID_CH), jnp.float32),
    ] + [pltpu.SemaphoreType.DMA] * (2 * NBUF),
    compiler_params=_sc_params,
)
def _sc_agg(src_hbm, dst_hbm, feat_hbm, zeros_hbm, out_hbm,
            srcv, dstv, rows, agg_sh, *sems):
    gsem = sems[:NBUF]
    ssem = sems[NBUF:]
    c = lax.axis_index("c")
    s = lax.axis_index("s")
    wid = s * NC + c
    r0 = s * RPT
    pltpu.sync_copy(zeros_hbm.at[pl.ds(r0, RPT)], agg_sh.at[pl.ds(r0, RPT)])
    pltpu.sync_copy(src_hbm.at[pl.ds(wid * CHUNKS, CHUNKS)], srcv)
    pltpu.sync_copy(dst_hbm.at[pl.ds(wid * CHUNKS, CHUNKS)], dstv)
    plsc.subcore_barrier()

    pltpu.async_copy(feat_hbm.at[srcv.at[0]], rows.at[0], gsem[0])
    pltpu.async_copy(feat_hbm.at[srcv.at[1]], rows.at[1], gsem[1])

    def body(i, carry):
        for b in range(NBUF):
            t = i * NBUF + b
            b2 = (b + 2) % NBUF
            pltpu.make_async_copy(feat_hbm.at[srcv.at[t]], rows.at[b],
                                  gsem[b]).wait()
            pltpu.async_copy(rows.at[b], agg_sh.at[dstv.at[t]], ssem[b],
                             add=True)

            @pl.when(t >= 2)
            def _():
                pltpu.make_async_copy(rows.at[b2], agg_sh.at[dstv.at[t]],
                                      ssem[b2]).wait()

            @pl.when(t + 2 < CHUNKS)
            def _():
                pltpu.async_copy(feat_hbm.at[srcv.at[t + 2]], rows.at[b2],
                                 gsem[b2])
        return carry

    lax.fori_loop(0, CHUNKS // NBUF, body, 0)
    pltpu.make_async_copy(rows.at[2], agg_sh.at[dstv.at[0]], ssem[2]).wait()
    pltpu.make_async_copy(rows.at[3], agg_sh.at[dstv.at[0]], ssem[3]).wait()
    plsc.subcore_barrier()
    pltpu.sync_copy(agg_sh.at[pl.ds(r0, RPT)],
                    out_hbm.at[pl.ds(c * N + r0, RPT)])


NV = N // 4
OUT_W = 8


def _tca_body(x_ref, w_ref, h_ref):
    w = w_ref[...]
    h_ref[...] = jnp.concatenate(
        [jnp.dot(x_ref[:, a * IN_CH:(a + 1) * IN_CH], w,
                 preferred_element_type=jnp.float32) for a in range(4)],
        axis=1)


def _tcb_body(h_ref, dg_ref, hs_ref, dinv_ref):
    deg = dg_ref[:NV] + dg_ref[NV:] + 1.0
    dinv = lax.rsqrt(deg)
    hs_ref[...] = h_ref[...] * dinv
    dinv_ref[...] = dinv


def _tc2_body(ag_ref, hs_ref, dinv_ref, b1_ref, zs_ref):
    dinv = dinv_ref[...]
    t = (ag_ref[:NV] + ag_ref[NV:] + hs_ref[...]) * dinv + b1_ref[...]
    zs_ref[...] = jnp.maximum(t, 0.0) * dinv


def _tc3_body(ag_ref, zs_ref, dinv_ref, w2_ref, b2_ref, out_ref):
    u = (ag_ref[:NV] + ag_ref[NV:] + zs_ref[...]) * dinv_ref[...]
    w2 = w2_ref[...]
    g = jnp.concatenate(
        [jnp.dot(u[:, a * HID_CH:(a + 1) * HID_CH], w2,
                 preferred_element_type=jnp.float32) for a in range(4)],
        axis=1)
    out_ref[...] = g + b2_ref[...]


def kernel(x, edge_index, W1, b1, W2, b2):
    ei = edge_index.astype(jnp.int32)
    npad = EPAD - N_EDGES
    src = jnp.concatenate(
        [ei[0], jnp.arange(npad, dtype=jnp.int32) % 8192])
    dst = jnp.concatenate(
        [ei[1], N + (jnp.arange(npad, dtype=jnp.int32) % 4096)])
    src = src.reshape(NW * CHUNKS, K)
    dst = dst.reshape(NW * CHUNKS, K)

    zeros_w = jnp.zeros((N, HID_CH), jnp.float32)
    zeros_d = jnp.zeros((N, DHW), jnp.float32)
    ones_k = jnp.ones((K, DHW), jnp.float32)
    b1t = jnp.tile(b1, 4).reshape(1, 128)
    b2t = jnp.tile(b2, 4).reshape(1, 8)
    xv = x.reshape(NV, 4 * IN_CH)

    deg2 = _sc_deg(dst, zeros_d, ones_k)
    deg2v = deg2.reshape(NC * N // 4, 128)

    h = pl.pallas_call(
        _tca_body,
        out_shape=jax.ShapeDtypeStruct((NV, 128), jnp.float32),
    )(xv, W1)

    hs, dinv = pl.pallas_call(
        _tcb_body,
        out_shape=[jax.ShapeDtypeStruct((NV, 128), jnp.float32),
                   jax.ShapeDtypeStruct((NV, 128), jnp.float32)],
    )(h, deg2v)

    agg1 = _sc_agg(src, dst, hs.reshape(N, HID_CH), zeros_w)
    agg1v = agg1.reshape(NC * N // 4, 128)

    zs = pl.pallas_call(
        _tc2_body,
        out_shape=jax.ShapeDtypeStruct((NV, 128), jnp.float32),
    )(agg1v, hs, dinv, b1t)

    agg2 = _sc_agg(src, dst, zs.reshape(N, HID_CH), zeros_w)
    agg2v = agg2.reshape(NC * N // 4, 128)

    outp = pl.pallas_call(
        _tc3_body,
        out_shape=jax.ShapeDtypeStruct((NV, 8), jnp.float32),
    )(agg2v, zs, dinv, W2, b2t)

    return outp.reshape(N, 2)


# speedup vs baseline: 1.7835x; 1.0684x over previous
import functools

import jax
import jax.numpy as jnp
from jax import lax
from jax.experimental import pallas as pl
from jax.experimental.pallas import tpu as pltpu
from jax.experimental.pallas import tpu_sc as plsc

N = 10000
IN_CH = 256
HID_CH = 32
N_EDGES = 160000

NC, NS = 2, 16
NW = NC * NS
RPT = N // NS
K = 128
CHUNKS = 40
EPAD = NW * CHUNKS * K
NSP = N + 4096
NBUF = 8
LOOK = NBUF // 2

_mesh = plsc.VectorSubcoreMesh(core_axis_name="c", subcore_axis_name="s")
_sc_params = pltpu.CompilerParams(use_tc_tiling_on_sc=False)


DHW = 16


@functools.partial(
    pl.kernel,
    mesh=_mesh,
    out_type=jax.ShapeDtypeStruct((NC * N, HID_CH), jnp.float32),
    scratch_types=[
        pltpu.VMEM((CHUNKS, K), jnp.int32),
        pltpu.VMEM((K, DHW), jnp.float32),
        pltpu.VMEM_SHARED((NSP, DHW), jnp.float32),
        pltpu.SemaphoreType.DMA,
    ],
    compiler_params=_sc_params,
)
def _sc_deg(dst_hbm, zeros_hbm, ones_hbm, out_hbm, dstv, onesv, deg_sh, sem):
    c = lax.axis_index("c")
    s = lax.axis_index("s")
    wid = s * NC + c
    r0 = s * RPT
    pltpu.sync_copy(zeros_hbm.at[pl.ds(r0, RPT)], deg_sh.at[pl.ds(r0, RPT)])
    pltpu.sync_copy(ones_hbm, onesv)
    pltpu.sync_copy(dst_hbm.at[pl.ds(wid * CHUNKS, CHUNKS)], dstv)
    plsc.subcore_barrier()

    def fire(j, carry):
        pltpu.async_copy(onesv, deg_sh.at[dstv.at[j]], sem, add=True)
        return carry

    lax.fori_loop(0, CHUNKS, fire, 0)

    def drain(j, carry):
        pltpu.make_async_copy(onesv, deg_sh.at[dstv.at[0]], sem).wait()
        return carry

    lax.fori_loop(0, CHUNKS, drain, 0)
    plsc.subcore_barrier()
    pltpu.sync_copy(deg_sh.at[pl.ds(r0, RPT)],
                    out_hbm.at[pl.ds(c * N + r0, RPT), pl.ds(0, DHW)])
    pltpu.sync_copy(deg_sh.at[pl.ds(r0, RPT)],
                    out_hbm.at[pl.ds(c * N + r0, RPT), pl.ds(DHW, DHW)])


@functools.partial(
    pl.kernel,
    mesh=_mesh,
    out_type=jax.ShapeDtypeStruct((NC * N, HID_CH), jnp.float32),
    scratch_types=[
        pltpu.VMEM((CHUNKS, K), jnp.int32),
        pltpu.VMEM((CHUNKS, K), jnp.int32),
        pltpu.VMEM((NBUF, K, HID_CH), jnp.float32),
        pltpu.VMEM_SHARED((NSP, HID_CH), jnp.float32),
    ] + [pltpu.SemaphoreType.DMA] * (2 * NBUF),
    compiler_params=_sc_params,
)
def _sc_agg(src_hbm, dst_hbm, feat_hbm, zeros_hbm, out_hbm,
            srcv, dstv, rows, agg_sh, *sems):
    gsem = sems[:NBUF]
    ssem = sems[NBUF:]
    c = lax.axis_index("c")
    s = lax.axis_index("s")
    wid = s * NC + c
    r0 = s * RPT
    pltpu.sync_copy(zeros_hbm.at[pl.ds(r0, RPT)], agg_sh.at[pl.ds(r0, RPT)])
    pltpu.sync_copy(src_hbm.at[pl.ds(wid * CHUNKS, CHUNKS)], srcv)
    pltpu.sync_copy(dst_hbm.at[pl.ds(wid * CHUNKS, CHUNKS)], dstv)
    plsc.subcore_barrier()

    for b in range(LOOK):
        pltpu.async_copy(feat_hbm.at[srcv.at[b]], rows.at[b], gsem[b])

    def body(i, carry):
        for b in range(NBUF):
            t = i * NBUF + b
            b2 = (b + LOOK) % NBUF
            pltpu.make_async_copy(feat_hbm.at[srcv.at[t]], rows.at[b],
                                  gsem[b]).wait()
            pltpu.async_copy(rows.at[b], agg_sh.at[dstv.at[t]], ssem[b],
                             add=True)

            @pl.when(t >= LOOK)
            def _():
                pltpu.make_async_copy(rows.at[b2], agg_sh.at[dstv.at[t]],
                                      ssem[b2]).wait()

            @pl.when(t + LOOK < CHUNKS)
            def _():
                pltpu.async_copy(feat_hbm.at[srcv.at[t + LOOK]], rows.at[b2],
                                 gsem[b2])
        return carry

    lax.fori_loop(0, CHUNKS // NBUF, body, 0)
    for b in range(LOOK):
        bb = (CHUNKS - LOOK + b) % NBUF
        pltpu.make_async_copy(rows.at[bb], agg_sh.at[dstv.at[0]],
                              ssem[bb]).wait()
    plsc.subcore_barrier()
    pltpu.sync_copy(agg_sh.at[pl.ds(r0, RPT)],
                    out_hbm.at[pl.ds(c * N + r0, RPT)])


NV = N // 4
OUT_W = 8


def _tca_body(x_ref, w_ref, h_ref):
    w = w_ref[...]
    h_ref[...] = jnp.concatenate(
        [jnp.dot(x_ref[:, a * IN_CH:(a + 1) * IN_CH], w,
                 preferred_element_type=jnp.float32) for a in range(4)],
        axis=1)


def _tcb_body(h_ref, dg_ref, hs_ref, dinv_ref):
    deg = dg_ref[:NV] + dg_ref[NV:] + 1.0
    dinv = lax.rsqrt(deg)
    hs_ref[...] = h_ref[...] * dinv
    dinv_ref[...] = dinv


def _tc2_body(ag_ref, hs_ref, dinv_ref, b1_ref, zs_ref):
    dinv = dinv_ref[...]
    t = (ag_ref[:NV] + ag_ref[NV:] + hs_ref[...]) * dinv + b1_ref[...]
    zs_ref[...] = jnp.maximum(t, 0.0) * dinv


def _tc3_body(ag_ref, zs_ref, dinv_ref, w2_ref, b2_ref, out_ref):
    u = (ag_ref[:NV] + ag_ref[NV:] + zs_ref[...]) * dinv_ref[...]
    w2 = w2_ref[...]
    g = jnp.concatenate(
        [jnp.dot(u[:, a * HID_CH:(a + 1) * HID_CH], w2,
                 preferred_element_type=jnp.float32) for a in range(4)],
        axis=1)
    out_ref[...] = g + b2_ref[...]


def kernel(x, edge_index, W1, b1, W2, b2):
    ei = edge_index.astype(jnp.int32).reshape(2 * N_EDGES)
    npad = EPAD - N_EDGES
    src = jnp.concatenate(
        [ei[:N_EDGES], jnp.arange(npad, dtype=jnp.int32) % 8192])
    dst = jnp.concatenate(
        [ei[N_EDGES:], N + (jnp.arange(npad, dtype=jnp.int32) % 4096)])
    src = src.reshape(NW * CHUNKS, K)
    dst = dst.reshape(NW * CHUNKS, K)

    zeros_w = jnp.zeros((N, HID_CH), jnp.float32)
    zeros_d = jnp.zeros((N, DHW), jnp.float32)
    ones_k = jnp.ones((K, DHW), jnp.float32)
    b1t = jnp.tile(b1, 4).reshape(1, 128)
    b2t = jnp.tile(b2, 4).reshape(1, 8)
    xv = x.reshape(NV, 4 * IN_CH)

    deg2 = _sc_deg(dst, zeros_d, ones_k)
    deg2v = deg2.reshape(NC * N // 4, 128)

    h = pl.pallas_call(
        _tca_body,
        out_shape=jax.ShapeDtypeStruct((NV, 128), jnp.float32),
    )(xv, W1)

    hs, dinv = pl.pallas_call(
        _tcb_body,
        out_shape=[jax.ShapeDtypeStruct((NV, 128), jnp.float32),
                   jax.ShapeDtypeStruct((NV, 128), jnp.float32)],
    )(h, deg2v)

    agg1 = _sc_agg(src, dst, hs.reshape(N, HID_CH), zeros_w)
    agg1v = agg1.reshape(NC * N // 4, 128)

    zs = pl.pallas_call(
        _tc2_body,
        out_shape=jax.ShapeDtypeStruct((NV, 128), jnp.float32),
    )(agg1v, hs, dinv, b1t)

    agg2 = _sc_agg(src, dst, zs.reshape(N, HID_CH), zeros_w)
    agg2v = agg2.reshape(NC * N // 4, 128)

    outp = pl.pallas_call(
        _tc3_body,
        out_shape=jax.ShapeDtypeStruct((NV, 8), jnp.float32),
    )(agg2v, zs, dinv, W2, b2t)

    return outp.reshape(N, 2)
